# baseline (speedup 1.0000x reference)
import functools
import math

import jax
import jax.numpy as jnp
from jax.experimental import pallas as pl
from jax.experimental.pallas import tpu as pltpu

TOKEN_BLOCK = 512


def _moe_kernel(x_ref, wg_ref, bg_ref, we_ref, be_ref, out_ref, *, n_experts):
    x = x_ref[...]
    logits = jnp.dot(x, wg_ref[...], preferred_element_type=jnp.float32)
    logits = logits + bg_ref[...]
    logits = logits - jnp.max(logits, axis=1, keepdims=True)
    p = jnp.exp(logits)
    w = p / jnp.sum(p, axis=1, keepdims=True)

    acc = jnp.zeros(out_ref.shape, dtype=jnp.float32)
    for e in range(n_experts):
        h = jnp.dot(x, we_ref[e], preferred_element_type=jnp.float32)
        h = h + be_ref[e]
        h = 0.5 * h * (1.0 + jax.lax.erf(h * (1.0 / math.sqrt(2.0))))
        acc = acc + w[:, e:e + 1] * h
    out_ref[...] = acc


def kernel(x, Wg, bg, We, be):
    T, D_IN = x.shape
    E = We.shape[0]
    D_OUT = We.shape[2]
    bt = min(TOKEN_BLOCK, T)
    grid = (T // bt,)

    return pl.pallas_call(
        functools.partial(_moe_kernel, n_experts=E),
        grid=grid,
        in_specs=[
            pl.BlockSpec((bt, D_IN), lambda i: (i, 0)),
            pl.BlockSpec((D_IN, E), lambda i: (0, 0)),
            pl.BlockSpec((1, E), lambda i: (0, 0)),
            pl.BlockSpec((E, D_IN, D_OUT), lambda i: (0, 0, 0)),
            pl.BlockSpec((E, D_OUT), lambda i: (0, 0)),
        ],
        out_specs=pl.BlockSpec((bt, D_OUT), lambda i: (i, 0)),
        out_shape=jax.ShapeDtypeStruct((T, D_OUT), jnp.float32),
    )(x.astype(jnp.bfloat16), Wg.astype(jnp.bfloat16), bg.reshape(1, E),
      We.astype(jnp.bfloat16), be)


# speedup vs baseline: 1.2107x; 1.2107x over previous
import functools
import math

import jax
import jax.numpy as jnp
from jax.experimental import pallas as pl
from jax.experimental.pallas import tpu as pltpu

TOKEN_BLOCK = 1024


def _moe_kernel(x_ref, wg_ref, bg_ref, we_ref, be_ref, out_ref, *, n_experts):
    x = x_ref[...]
    logits = jnp.dot(x, wg_ref[...], preferred_element_type=jnp.float32)
    logits = logits + bg_ref[...]
    logits = logits - jnp.max(logits, axis=1, keepdims=True)
    p = jnp.exp(logits)
    w = p / jnp.sum(p, axis=1, keepdims=True)

    acc = jnp.zeros(out_ref.shape, dtype=jnp.float32)
    for e in range(n_experts):
        h = jnp.dot(x, we_ref[e], preferred_element_type=jnp.float32)
        h = h + be_ref[e]
        h = 0.5 * h * (1.0 + jax.lax.erf(h * (1.0 / math.sqrt(2.0))))
        acc = acc + w[:, e:e + 1] * h
    out_ref[...] = acc


def kernel(x, Wg, bg, We, be):
    T, D_IN = x.shape
    E = We.shape[0]
    D_OUT = We.shape[2]
    bt = min(TOKEN_BLOCK, T)
    grid = (T // bt,)

    return pl.pallas_call(
        functools.partial(_moe_kernel, n_experts=E),
        grid=grid,
        in_specs=[
            pl.BlockSpec((bt, D_IN), lambda i: (i, 0)),
            pl.BlockSpec((D_IN, E), lambda i: (0, 0)),
            pl.BlockSpec((1, E), lambda i: (0, 0)),
            pl.BlockSpec((E, D_IN, D_OUT), lambda i: (0, 0, 0)),
            pl.BlockSpec((E, D_OUT), lambda i: (0, 0)),
        ],
        out_specs=pl.BlockSpec((bt, D_OUT), lambda i: (i, 0)),
        out_shape=jax.ShapeDtypeStruct((T, D_OUT), jnp.float32),
    )(x, Wg, bg.reshape(1, E), We, be)
